# baseline (speedup 1.0000x reference)
import functools

import jax
import jax.numpy as jnp
from jax import lax
from jax.experimental import pallas as pl
from jax.experimental.pallas import tpu as pltpu
from jax.experimental.pallas import tpu_sc as plsc

NUM_TOKENS = 1000000
DIM = 64
BATCH = 4096
SEQ = 200

NC = 2
NS = 16
NW = NC * NS

TOTAL = BATCH * SEQ
IPG = 128
NIROW = TOTAL // IPG
IR_PW = NIROW // NW
BBLK = BATCH // IPG
DEPTH = 4

_mesh = plsc.VectorSubcoreMesh(core_axis_name="c", subcore_axis_name="s")


@functools.partial(
    pl.kernel,
    out_type=jax.ShapeDtypeStruct((SEQ, DIM // 8, BBLK, 8, IPG), jnp.float32),
    mesh=_mesh,
    scratch_types=[
        pltpu.VMEM((IR_PW, IPG), jnp.int32),
        [pltpu.VMEM((IPG, 2 * DIM), jnp.float32) for _ in range(DEPTH)],
        [pltpu.VMEM((DIM // 8, 1, 8, IPG + 1), jnp.float32)
         for _ in range(DEPTH)],
        [pltpu.SemaphoreType.DMA for _ in range(DEPTH)],
        [pltpu.SemaphoreType.DMA for _ in range(DEPTH)],
    ],
    compiler_params=pltpu.CompilerParams(
        use_tc_tiling_on_sc=False, needs_layout_passes=False
    ),
)
def _emb_lookup(table_hbm, idx_hbm, out_hbm, idx_v, bufs, bts, gsems, osems):
    wid = lax.axis_index("s") * NC + lax.axis_index("c")
    c_base = wid * IR_PW
    pltpu.sync_copy(idx_hbm.at[pl.ds(c_base, IR_PW)], idx_v)

    dt_idx = [lax.iota(jnp.int32, 16) // 8 + 2 * dg for dg in range(4)]
    d8_idx = lax.rem(lax.iota(jnp.int32, 16), 8)
    zero = jnp.zeros((16,), jnp.int32)

    def fire_gather(u, buf, gsem):
        pltpu.async_copy(table_hbm.at[idx_v.at[u]], buf, gsem)

    def wait_gather(buf, gsem):
        pltpu.make_async_copy(table_hbm.at[idx_v.at[0]], buf, gsem).wait()

    def transpose(buf, bt):
        @plsc.parallel_loop(0, IPG)
        def _(b):
            bcol = jnp.full((16,), b, jnp.int32)
            for dg in range(4):
                v = buf[b, pl.ds(16 * dg, 16)]
                plsc.store_scatter(bt, [dt_idx[dg], zero, d8_idx, bcol], v)

    def fire_out(u, bt, osem):
        c = c_base + u
        s = c // BBLK
        bc = lax.rem(c, BBLK)
        pltpu.async_copy(bt.at[:, :, :, pl.ds(0, IPG)],
                         out_hbm.at[s, :, pl.ds(bc, 1)], osem)

    def wait_out(bt, osem):
        pltpu.make_async_copy(bt.at[:, :, :, pl.ds(0, IPG)],
                              out_hbm.at[0, :, pl.ds(0, 1)], osem).wait()

    for r in range(DEPTH):
        fire_gather(r, bufs[r], gsems[r])

    def body(t, carry):
        for r in range(DEPTH):
            u = DEPTH * t + r
            wait_gather(bufs[r], gsems[r])

            @pl.when(t > 0)
            def _():
                wait_out(bts[r], osems[r])

            transpose(bufs[r], bts[r])
            fire_out(u, bts[r], osems[r])

            @pl.when(u + DEPTH < IR_PW)
            def _():
                fire_gather(u + DEPTH, bufs[r], gsems[r])

        return carry

    lax.fori_loop(0, IR_PW // DEPTH, body, 0)
    for r in range(DEPTH):
        wait_out(bts[r], osems[r])


def kernel(x, emb_weight):
    idx = x.T.reshape(NIROW, IPG)
    emb_p = jnp.pad(emb_weight, ((0, 0), (0, DIM)))
    out5 = _emb_lookup(emb_p, idx)
    return out5.transpose(2, 4, 0, 1, 3).reshape(BATCH, SEQ, DIM)


# speedup vs baseline: 1.0117x; 1.0117x over previous
import functools

import jax
import jax.numpy as jnp
from jax import lax
from jax.experimental import pallas as pl
from jax.experimental.pallas import tpu as pltpu
from jax.experimental.pallas import tpu_sc as plsc

NUM_TOKENS = 1000000
DIM = 64
BATCH = 4096
SEQ = 200

NC = 2
NS = 16
NW = NC * NS

TOTAL = BATCH * SEQ
IPG = 128
NIROW = TOTAL // IPG
IR_PW = NIROW // NW
BBLK = BATCH // IPG
DEPTH = 4

_mesh = plsc.VectorSubcoreMesh(core_axis_name="c", subcore_axis_name="s")


@functools.partial(
    pl.kernel,
    out_type=jax.ShapeDtypeStruct((SEQ, DIM // 8, BBLK, 8, IPG), jnp.float32),
    mesh=_mesh,
    scratch_types=[
        pltpu.VMEM((IR_PW, IPG), jnp.int32),
        [pltpu.VMEM((IPG, DIM), jnp.float32) for _ in range(DEPTH)],
        [pltpu.VMEM((DIM // 8, 1, 8, IPG + 1), jnp.float32)
         for _ in range(DEPTH)],
        [pltpu.SemaphoreType.DMA for _ in range(DEPTH)],
        [pltpu.SemaphoreType.DMA for _ in range(DEPTH)],
    ],
    compiler_params=pltpu.CompilerParams(
        use_tc_tiling_on_sc=False, needs_layout_passes=False
    ),
)
def _emb_lookup(table_hbm, idx_hbm, out_hbm, idx_v, bufs, bts, gsems, osems):
    wid = lax.axis_index("s") * NC + lax.axis_index("c")
    c_base = wid * IR_PW
    pltpu.sync_copy(idx_hbm.at[pl.ds(c_base, IR_PW)], idx_v)

    dt_idx = [lax.iota(jnp.int32, 16) // 8 + 2 * dg for dg in range(4)]
    d8_idx = lax.rem(lax.iota(jnp.int32, 16), 8)
    zero = jnp.zeros((16,), jnp.int32)

    def fire_gather(u, buf, gsem):
        pltpu.async_copy(table_hbm.at[idx_v.at[u]], buf, gsem)

    def wait_gather(buf, gsem):
        pltpu.make_async_copy(table_hbm.at[idx_v.at[0]], buf, gsem).wait()

    def transpose(buf, bt):
        @plsc.parallel_loop(0, IPG)
        def _(b):
            bcol = jnp.full((16,), b, jnp.int32)
            for dg in range(4):
                v = buf[b, pl.ds(16 * dg, 16)]
                plsc.store_scatter(bt, [dt_idx[dg], zero, d8_idx, bcol], v)

    def fire_out(u, bt, osem):
        c = c_base + u
        s = c // BBLK
        bc = lax.rem(c, BBLK)
        pltpu.async_copy(bt.at[:, :, :, pl.ds(0, IPG)],
                         out_hbm.at[s, :, pl.ds(bc, 1)], osem)

    def wait_out(bt, osem):
        pltpu.make_async_copy(bt.at[:, :, :, pl.ds(0, IPG)],
                              out_hbm.at[0, :, pl.ds(0, 1)], osem).wait()

    for r in range(DEPTH):
        fire_gather(r, bufs[r], gsems[r])

    def body(t, carry):
        for r in range(DEPTH):
            u = DEPTH * t + r
            wait_gather(bufs[r], gsems[r])

            @pl.when(t > 0)
            def _():
                wait_out(bts[r], osems[r])

            transpose(bufs[r], bts[r])
            fire_out(u, bts[r], osems[r])

            @pl.when(u + DEPTH < IR_PW)
            def _():
                fire_gather(u + DEPTH, bufs[r], gsems[r])

        return carry

    lax.fori_loop(0, IR_PW // DEPTH, body, 0)
    for r in range(DEPTH):
        wait_out(bts[r], osems[r])


def kernel(x, emb_weight):
    idx = x.T.reshape(NIROW, IPG)
    out5 = _emb_lookup(emb_weight, idx)
    return out5.transpose(2, 4, 0, 1, 3).reshape(BATCH, SEQ, DIM)
